# baseline (speedup 1.0000x reference)
import functools

import jax
import jax.numpy as jnp
from jax import lax
from jax.experimental import pallas as pl
from jax.experimental.pallas import tpu as pltpu
from jax.experimental.pallas import tpu_sc as plsc

D_MODEL = 1024
NUM_CORES = 2
NUM_SUBCORES = 16
NUM_WORKERS = NUM_CORES * NUM_SUBCORES


@functools.lru_cache(maxsize=None)
def _make_gather(B: int, C: int, NBUF: int):
    b_per_w = B // NUM_WORKERS
    n_chunks = b_per_w // C
    mesh = plsc.VectorSubcoreMesh(
        core_axis_name="c",
        subcore_axis_name="s",
        num_cores=NUM_CORES,
        num_subcores=NUM_SUBCORES,
    )

    @functools.partial(
        pl.kernel,
        out_type=jax.ShapeDtypeStruct((B, D_MODEL), jnp.float32),
        mesh=mesh,
        scratch_types=[
            pltpu.VMEM((b_per_w,), jnp.int32),
            pltpu.VMEM((NBUF, C, D_MODEL), jnp.float32),
            [pltpu.SemaphoreType.DMA] * NBUF,
            [pltpu.SemaphoreType.DMA] * NBUF,
        ],
    )
    def gather_kernel(table_hbm, idx_hbm, out_hbm, idx_v, rows, gsems, wsems):
        wid = lax.axis_index("s") * NUM_CORES + lax.axis_index("c")
        base = wid * b_per_w
        pltpu.sync_copy(idx_hbm.at[pl.ds(base, b_per_w)], idx_v)

        def start_gather(c):
            b = c % NBUF
            return pltpu.async_copy(
                table_hbm.at[idx_v.at[pl.ds(c * C, C)]], rows.at[b], gsems[b]
            )

        gops = [None] * n_chunks
        wops = [None] * n_chunks
        for c in range(min(NBUF, n_chunks)):
            gops[c] = start_gather(c)
        for c in range(n_chunks):
            b = c % NBUF
            if c >= NBUF:
                wops[c - NBUF].wait()
                gops[c] = start_gather(c)
            gops[c].wait()
            wops[c] = pltpu.async_copy(
                rows.at[b], out_hbm.at[pl.ds(base + c * C, C)], wsems[b]
            )
        for c in range(max(0, n_chunks - NBUF), n_chunks):
            wops[c].wait()

    return gather_kernel


def kernel(x, p2e):
    shp = x.shape
    idx = x.reshape(-1).astype(jnp.int32)
    out = _make_gather(idx.shape[0], 32, 3)(p2e, idx)
    return out.reshape(shp + (D_MODEL,))
